# baseline (speedup 1.0000x reference)
import functools

import jax
import jax.numpy as jnp
from jax import lax
from jax.experimental import pallas as pl
from jax.experimental import pallas as _pl_unused
from jax.experimental.pallas import tpu as pltpu
from jax.experimental.pallas import tpu_sc as plsc

_L = 16
_NC = 2
_NS = 16
_NW = _NC * _NS
_CH = 64


def _maskout_body(bpw, d, x_hbm, label_hbm, out_hbm, label_v, rows3_v, out_v, sems):
    cid = lax.axis_index("c")
    sid = lax.axis_index("s")
    wid = sid * _NC + cid
    base = wid * bpw

    pltpu.sync_copy(label_hbm.at[pl.ds(base, bpw)], label_v.at[pl.ds(0, bpw)])

    n_chunks = bpw // _CH
    copies = [None, None]
    copies[0] = pltpu.async_copy(
        x_hbm.at[pl.ds(base, _CH)], rows3_v.at[0], sems.at[0]
    )
    for k in range(n_chunks):
        par = k % 2
        if k + 1 < n_chunks:
            copies[(k + 1) % 2] = pltpu.async_copy(
                x_hbm.at[pl.ds(base + (k + 1) * _CH, _CH)],
                rows3_v.at[(k + 1) % 2],
                sems.at[(k + 1) % 2],
            )
        copies[par].wait()

        def select(j, _, k=k, par=par):
            lbl = label_v[pl.ds(k * _CH + j, _L)][0]
            for c8 in range(d // _L):
                out_v[par, j, pl.ds(c8 * _L, _L)] = rows3_v[
                    par, j, lbl, pl.ds(c8 * _L, _L)
                ]
            return 0

        lax.fori_loop(0, _CH, select, 0)
        pltpu.sync_copy(out_v.at[par], out_hbm.at[pl.ds(base + k * _CH, _CH)])


@jax.jit
def kernel(x, label):
    batch, nr_cate, d = x.shape
    bpw = batch // _NW

    mesh = plsc.VectorSubcoreMesh(core_axis_name="c", subcore_axis_name="s")
    run = pl.kernel(
        functools.partial(_maskout_body, bpw, d),
        out_type=jax.ShapeDtypeStruct((batch, d), x.dtype),
        mesh=mesh,
        scratch_types=[
            pltpu.VMEM((bpw + _L,), jnp.int32),
            pltpu.VMEM((2, _CH, nr_cate, d), jnp.float32),
            pltpu.VMEM((2, _CH, d), jnp.float32),
            pltpu.SemaphoreType.DMA((2,)),
        ],
    )
    return run(x, label)
